# baseline (speedup 1.0000x reference)
import functools
import jax
import jax.numpy as jnp
from jax import lax
from jax.experimental import pallas as pl
from jax.experimental.pallas import tpu as pltpu
from jax.experimental.pallas import tpu_sc as plsc

B, N_H, N_PTS, K_SEL, NEIGHS = 2, 64, 4096, 1024, 20
EPS = 1e-5
HI = jax.lax.Precision.HIGHEST
RBLK = 256
NBLK = N_PTS // RBLK


def _knn_body(h1_ref, xx_ref, out_ref):
    b = pl.program_id(0)
    i = pl.program_id(1)
    hb = h1_ref[0].astype(jnp.bfloat16)
    rows = h1_ref[0, :, pl.ds(i * RBLK, RBLK)].astype(jnp.bfloat16)
    A = jax.lax.dot_general(rows, hb, (((0,), (0,)), ((), ())),
                            preferred_element_type=jnp.float32)
    xx = xx_ref[0]
    xr = jnp.reshape(xx_ref[0, 0, pl.ds(i * RBLK, RBLK)], (RBLK, 1))
    P = 2.0 * A - xr - xx
    iota_j = jax.lax.broadcasted_iota(jnp.int32, (RBLK, N_PTS), 1)
    for k in range(NEIGHS):
        m = jnp.max(P, axis=1, keepdims=True)
        cand = jnp.where(P == m, iota_j, N_PTS)
        am = jnp.min(cand, axis=1)
        out_ref[0, k, :] = am
        P = jnp.where(cand == am[:, None], -jnp.inf, P)


def _knn_topk(h1, xx):
    return pl.pallas_call(
        _knn_body,
        grid=(B, NBLK),
        in_specs=[
            pl.BlockSpec((1, N_H, N_PTS), lambda b, i: (b, 0, 0)),
            pl.BlockSpec((1, 1, N_PTS), lambda b, i: (b, 0, 0)),
        ],
        out_specs=pl.BlockSpec((1, NEIGHS, RBLK), lambda b, i: (b, 0, i)),
        out_shape=jax.ShapeDtypeStruct((B, NEIGHS, N_PTS), jnp.int32),
    )(h1, xx.reshape(B, 1, N_PTS))


N_ROWS = B * N_PTS
NW = 32
PER_W = N_ROWS // NW
CHUNK = 128
NCH = PER_W // CHUNK


TW = 2 * N_H


@functools.partial(
    pl.kernel,
    mesh=plsc.VectorSubcoreMesh(core_axis_name="c", subcore_axis_name="s"),
    out_type=jax.ShapeDtypeStruct((N_ROWS, TW), jnp.float32),
    scratch_types=[
        pltpu.VMEM((CHUNK,), jnp.int32),
        pltpu.VMEM((CHUNK, TW), jnp.float32),
        pltpu.SemaphoreType.DMA,
    ],
)
def _sc_gather(table_hbm, idx_hbm, out_hbm, idx_v, rows_v, sem):
    wid = lax.axis_index("s") * 2 + lax.axis_index("c")
    base = wid * PER_W

    def chunk(j, carry):
        off = base + j * CHUNK
        pltpu.sync_copy(idx_hbm.at[pl.ds(off, CHUNK)], idx_v)
        pltpu.async_copy(table_hbm.at[idx_v], rows_v, sem).wait()
        pltpu.sync_copy(rows_v, out_hbm.at[pl.ds(off, CHUNK)])
        return carry

    lax.fori_loop(0, NCH, chunk, 0)


def kernel(xyz, seq1, W_fc, b_fc, g1, beta1, W_conv, g2, beta2, W_bil, b_bil, perm):
    A = jnp.einsum('oc,bcn->bon', W_fc, seq1) + b_fc[None, :, None]
    m1 = jnp.mean(A, axis=(0, 2), keepdims=True)
    v1 = jnp.var(A, axis=(0, 2), keepdims=True)
    h1 = jnp.maximum((A - m1) / jnp.sqrt(v1 + EPS) * g1.reshape(1, N_H, 1)
                     + beta1.reshape(1, N_H, 1), 0.0)

    xx = jnp.sum(h1 * h1, axis=1)
    idx20 = jnp.transpose(_knn_topk(h1, xx), (0, 2, 1))

    h1t = jnp.transpose(h1, (0, 2, 1))
    table = jnp.pad(h1t.reshape(B * N_PTS, N_H), ((0, 0), (0, TW - N_H)))
    permg = perm[None, :].astype(jnp.int32) + (jnp.arange(B, dtype=jnp.int32) * N_PTS)[:, None]
    h2t = _sc_gather(table, permg.reshape(-1))[:, :N_H].reshape(B, N_PTS, N_H)

    feat4 = jax.vmap(lambda t, i: t[i])(h1t, idx20)
    ctr4 = h1t[:, :, None, :]
    gf = jnp.concatenate([feat4 - ctr4, jnp.broadcast_to(ctr4, feat4.shape)], axis=3)
    gf = jnp.transpose(gf, (0, 3, 1, 2))
    hcs = jnp.einsum('oc,bcnk->bonk', W_conv, gf)
    hcb = ((hcs - jnp.mean(hcs, axis=(0, 2, 3), keepdims=True))
           / jnp.sqrt(jnp.var(hcs, axis=(0, 2, 3), keepdims=True) + EPS)
           * g2.reshape(1, N_H, 1, 1) + beta2.reshape(1, N_H, 1, 1))
    hcb = jnp.where(hcb > 0, hcb, 0.2 * hcb)
    h_n1 = jnp.max(hcb, axis=-1)
    Xt = jnp.transpose(jax.nn.sigmoid(h_n1), (0, 2, 1))

    T1 = jnp.einsum('bni,ij->bnj', h1t, W_bil)
    sc1 = jnp.sum(T1 * Xt, axis=2) + b_bil
    T2 = jnp.einsum('bni,ij->bnj', h2t, W_bil)
    sc2 = jnp.sum(T2 * Xt, axis=2) + b_bil
    logits = jnp.concatenate([sc1, sc2], axis=1)

    scores = jax.nn.sigmoid(sc1)
    gt = (scores[:, None, :] > scores[:, :, None]).astype(jnp.int32)
    iot = jnp.arange(N_PTS)
    tie = ((scores[:, None, :] == scores[:, :, None])
           & (iot[None, None, :] < iot[None, :, None])).astype(jnp.int32)
    rank = jnp.sum(gt + tie, axis=2)
    M = (rank[:, None, :] == jnp.arange(K_SEL)[None, :, None]).astype(jnp.float32)
    values = jnp.einsum('bri,bi->br', M, scores, precision=HI)
    idx = jnp.einsum('bri,i->br', M, iot.astype(jnp.float32), precision=HI).astype(jnp.int32)
    seq_static = jnp.einsum('bcn,brn->bcr', seq1, M, precision=HI)
    seq = seq_static * values[:, None, :]
    xyz_static = jnp.einsum('bcn,brn->bcr', xyz, M, precision=HI)
    xyz_out = xyz_static * values[:, None, :]

    return seq, values, idx, logits, xyz_static, xyz_out
